# baseline (speedup 1.0000x reference)
import jax
import jax.numpy as jnp
from jax.experimental import pallas as pl
from jax.experimental.pallas import tpu as pltpu

_GRAN = 64
_P = 32
_D = 2
_BB = 32


def _chamfer_body(n_ref, yc_ref, ext_ref, out_ref):
    b = pl.program_id(0)
    lane = jax.lax.broadcasted_iota(jnp.int32, (_P, 2 * _GRAN), 1)
    first_half = lane < _GRAN
    mins0, mins1 = [], []
    for t in range(_BB // 2):
        j0, j1 = 2 * t, 2 * t + 1
        i0 = n_ref[_BB * b + j0]
        i1 = n_ref[_BB * b + j1]
        ax = jnp.where(first_half, ext_ref[i0, 0], ext_ref[i1, 0])
        ay = jnp.where(first_half, ext_ref[i0, 1], ext_ref[i1, 1])
        a2 = ax * ax + ay * ay
        axm = ax.astype(jnp.bfloat16).astype(jnp.float32) * -2.0
        aym = ay.astype(jnp.bfloat16).astype(jnp.float32) * -2.0
        bx = jnp.where(first_half,
                       jnp.broadcast_to(yc_ref[j0, :, 0:1], (_P, 2 * _GRAN)),
                       jnp.broadcast_to(yc_ref[j1, :, 0:1], (_P, 2 * _GRAN)))
        by = jnp.where(first_half,
                       jnp.broadcast_to(yc_ref[j0, :, 1:2], (_P, 2 * _GRAN)),
                       jnp.broadcast_to(yc_ref[j1, :, 1:2], (_P, 2 * _GRAN)))
        b2 = bx * bx + by * by
        bxb = bx.astype(jnp.bfloat16).astype(jnp.float32)
        byb = by.astype(jnp.bfloat16).astype(jnp.float32)
        m1a = m1b = None
        s2a = s2b = None
        for q in range(_P):
            sl = ((a2 + b2[q:q + 1]) + axm * bxb[q:q + 1]
                  + aym * byb[q:q + 1])
            mq = jnp.min(sl, axis=0, keepdims=True)
            sq = jnp.sqrt(jnp.maximum(mq, 0.0) + 1e-12)
            if q % 2 == 0:
                m1a = sl if m1a is None else jnp.minimum(m1a, sl)
                s2a = sq if s2a is None else s2a + sq
            else:
                m1b = sl if m1b is None else jnp.minimum(m1b, sl)
                s2b = sq if s2b is None else s2b + sq
        m1 = jnp.minimum(m1a, m1b)
        s1 = jnp.mean(jnp.sqrt(jnp.maximum(m1, 0.0) + 1e-12), axis=0)
        s2 = (s2a + s2b)[0] * (1.0 / _P)
        tot = s1 + s2
        mins0.append(jnp.min(tot[:_GRAN]))
        mins1.append(jnp.min(tot[_GRAN:]))
    vals = []
    for t in range(_BB // 2):
        vals.extend([mins0[t], mins1[t]])
    out_ref[...] = jnp.stack(vals).reshape(_BB, 1, 1)


def kernel(y, n, labels, examples):
    del labels
    B = y.shape[0]
    C = examples.shape[0]
    e1 = jnp.swapaxes(examples.reshape(C, _GRAN, _P * _D), 1, 2)
    e3 = jnp.swapaxes(e1.reshape(C, _P, _D, _GRAN), 1, 2)
    exT2 = jnp.concatenate([e3, e3], axis=-1)
    grid_spec = pltpu.PrefetchScalarGridSpec(
        num_scalar_prefetch=1,
        grid=(B // _BB,),
        in_specs=[
            pl.BlockSpec((_BB, _P, _D), lambda b, n_ref: (b, 0, 0)),
            pl.BlockSpec((C, _D, _P, 2 * _GRAN), lambda b, n_ref: (0, 0, 0, 0)),
        ],
        out_specs=pl.BlockSpec((_BB, 1, 1), lambda b, n_ref: (b, 0, 0)),
    )
    out = pl.pallas_call(
        _chamfer_body,
        grid_spec=grid_spec,
        out_shape=jax.ShapeDtypeStruct((B, 1, 1), jnp.float32),
    )(n.astype(jnp.int32), y, exT2)
    return out[:, 0, 0]


# speedup vs baseline: 1.1418x; 1.1418x over previous
import jax
import jax.numpy as jnp
from jax.experimental import pallas as pl
from jax.experimental.pallas import tpu as pltpu

_GRAN = 64
_P = 32
_D = 2
_BB = 16


def _chamfer_body(n_ref, yc_ref, ext_ref, out_ref):
    b = pl.program_id(0)
    lane = jax.lax.broadcasted_iota(jnp.int32, (_P, 2 * _GRAN), 1)
    first_half = lane < _GRAN
    mins0, mins1 = [], []
    for t in range(_BB // 2):
        j0, j1 = 2 * t, 2 * t + 1
        i0 = n_ref[_BB * b + j0]
        i1 = n_ref[_BB * b + j1]
        ax = jnp.where(first_half, ext_ref[i0, 0], ext_ref[i1, 0])
        ay = jnp.where(first_half, ext_ref[i0, 1], ext_ref[i1, 1])
        a2 = ax * ax + ay * ay
        axm = ax.astype(jnp.bfloat16).astype(jnp.float32) * -2.0
        aym = ay.astype(jnp.bfloat16).astype(jnp.float32) * -2.0
        bx = jnp.where(first_half,
                       jnp.broadcast_to(yc_ref[j0, :, 0:1], (_P, 2 * _GRAN)),
                       jnp.broadcast_to(yc_ref[j1, :, 0:1], (_P, 2 * _GRAN)))
        by = jnp.where(first_half,
                       jnp.broadcast_to(yc_ref[j0, :, 1:2], (_P, 2 * _GRAN)),
                       jnp.broadcast_to(yc_ref[j1, :, 1:2], (_P, 2 * _GRAN)))
        b2 = bx * bx + by * by
        bxb = bx.astype(jnp.bfloat16).astype(jnp.float32)
        byb = by.astype(jnp.bfloat16).astype(jnp.float32)
        m1a = m1b = None
        partials = []
        for q in range(_P):
            sl = ((a2 + b2[q:q + 1]) + axm * bxb[q:q + 1]
                  + aym * byb[q:q + 1])
            partials.append(jnp.min(sl.reshape(4, 8, 2 * _GRAN), axis=0))
            if q % 2 == 0:
                m1a = sl if m1a is None else jnp.minimum(m1a, sl)
            else:
                m1b = sl if m1b is None else jnp.minimum(m1b, sl)
        m1 = jnp.minimum(m1a, m1b)
        s1 = jnp.mean(jnp.sqrt(jnp.maximum(m1, 0.0) + 1e-12), axis=0)
        m2rows = []
        for i in range(0, _P, 4):
            blk = jnp.stack(partials[i:i + 4])
            m2rows.append(jnp.min(blk, axis=1))
        m2 = jnp.concatenate(m2rows, axis=0)
        s2 = jnp.mean(jnp.sqrt(jnp.maximum(m2, 0.0) + 1e-12), axis=0)
        tot = s1 + s2
        mins0.append(jnp.min(tot[:_GRAN]))
        mins1.append(jnp.min(tot[_GRAN:]))
    vals = []
    for t in range(_BB // 2):
        vals.extend([mins0[t], mins1[t]])
    out_ref[...] = jnp.stack(vals).reshape(_BB, 1, 1)


def kernel(y, n, labels, examples):
    del labels
    B = y.shape[0]
    C = examples.shape[0]
    e1 = jnp.swapaxes(examples.reshape(C, _GRAN, _P * _D), 1, 2)
    e3 = jnp.swapaxes(e1.reshape(C, _P, _D, _GRAN), 1, 2)
    exT2 = jnp.concatenate([e3, e3], axis=-1)
    grid_spec = pltpu.PrefetchScalarGridSpec(
        num_scalar_prefetch=1,
        grid=(B // _BB,),
        in_specs=[
            pl.BlockSpec((_BB, _P, _D), lambda b, n_ref: (b, 0, 0)),
            pl.BlockSpec((C, _D, _P, 2 * _GRAN), lambda b, n_ref: (0, 0, 0, 0)),
        ],
        out_specs=pl.BlockSpec((_BB, 1, 1), lambda b, n_ref: (b, 0, 0)),
    )
    out = pl.pallas_call(
        _chamfer_body,
        grid_spec=grid_spec,
        out_shape=jax.ShapeDtypeStruct((B, 1, 1), jnp.float32),
    )(n.astype(jnp.int32), y, exT2)
    return out[:, 0, 0]
